# Initial kernel scaffold; baseline (speedup 1.0000x reference)
import functools

import jax
import jax.numpy as jnp
from jax import lax
from jax.experimental import pallas as pl
from jax.experimental.pallas import tpu as pltpu
from jax.experimental.pallas import tpu_sc as plsc

D = 32
B_TOTAL = 4096 * 200
NC = 2
NS = 16
NW = NC * NS
B_PER_W = B_TOTAL // NW
CHUNK = 1024
N_CHUNKS = B_PER_W // CHUNK


def _emb_body(x_hbm, w_hbm, out_hbm, idx_v, rows_v, sem):
    wid = lax.axis_index("s") * NC + lax.axis_index("c")
    base = wid * B_PER_W

    def chunk(j, carry):
        off = base + j * CHUNK
        pltpu.sync_copy(x_hbm.at[pl.ds(off, CHUNK)], idx_v)
        pltpu.async_copy(w_hbm.at[idx_v], rows_v, sem).wait()
        pltpu.sync_copy(rows_v, out_hbm.at[pl.ds(off, CHUNK)])
        return carry

    lax.fori_loop(0, N_CHUNKS, chunk, 0)


@jax.jit
def _emb_call(x_flat, weight):
    fn = functools.partial(
        pl.kernel,
        mesh=plsc.VectorSubcoreMesh(core_axis_name="c", subcore_axis_name="s"),
        out_type=jax.ShapeDtypeStruct((B_TOTAL, D), jnp.float32),
        scratch_types=[
            pltpu.VMEM((CHUNK,), jnp.int32),
            pltpu.VMEM((CHUNK, D), jnp.float32),
            pltpu.SemaphoreType.DMA,
        ],
    )(_emb_body)
    return fn(x_flat, weight)


def kernel(x, weight):
    x_flat = x.reshape(-1).astype(jnp.int32)
    out = _emb_call(x_flat, weight)
    return out.reshape(x.shape + (weight.shape[1],))


# speedup vs baseline: 1.4596x; 1.4596x over previous
import functools

import jax
import jax.numpy as jnp
from jax import lax
from jax.experimental import pallas as pl
from jax.experimental.pallas import tpu as pltpu
from jax.experimental.pallas import tpu_sc as plsc

D = 32
B_TOTAL = 4096 * 200
NC = 2
NS = 16
NW = NC * NS
B_PER_W = B_TOTAL // NW
CHUNK = 1024
N_CHUNKS = B_PER_W // CHUNK


def _emb_body(x_hbm, w_hbm, out_hbm, idx_v, rows_v, sem):
    wid = lax.axis_index("s") * NC + lax.axis_index("c")
    base = wid * B_PER_W

    def chunk(j, carry):
        off = base + j * CHUNK
        pltpu.sync_copy(x_hbm.at[pl.ds(off, CHUNK)], idx_v)
        pltpu.async_copy(w_hbm.at[idx_v], rows_v, sem).wait()
        pltpu.sync_copy(rows_v, out_hbm.at[pl.ds(off, CHUNK)])
        return carry

    lax.fori_loop(0, N_CHUNKS, chunk, 0)


@jax.jit
def _emb_call(x_flat, weight):
    fn = functools.partial(
        pl.kernel,
        mesh=plsc.VectorSubcoreMesh(core_axis_name="c", subcore_axis_name="s"),
        out_type=jax.ShapeDtypeStruct((B_TOTAL, D), jnp.float32),
        scratch_types=[
            pltpu.VMEM((CHUNK,), jnp.int32),
            pltpu.VMEM((CHUNK, D), jnp.float32),
            pltpu.SemaphoreType.DMA,
        ],
        compiler_params=pltpu.CompilerParams(use_tc_tiling_on_sc=False),
    )(_emb_body)
    return fn(x_flat, weight)


def kernel(x, weight):
    x_flat = x.reshape(-1).astype(jnp.int32)
    out = _emb_call(x_flat, weight)
    return out.reshape(x.shape + (weight.shape[1],))


# speedup vs baseline: 1.5002x; 1.0278x over previous
import functools

import jax
import jax.numpy as jnp
from jax import lax
from jax.experimental import pallas as pl
from jax.experimental.pallas import tpu as pltpu
from jax.experimental.pallas import tpu_sc as plsc

D = 32
B_TOTAL = 4096 * 200
NC = 2
NS = 16
NW = NC * NS
B_PER_W = B_TOTAL // NW
CHUNK = 1280
N_CHUNKS = B_PER_W // CHUNK


def _emb_body(x_hbm, w_hbm, out_hbm, idx_all, rows0, rows1,
              sg0, sg1, sw0, sw1):
    wid = lax.axis_index("s") * NC + lax.axis_index("c")
    base = wid * B_PER_W
    pltpu.sync_copy(x_hbm.at[pl.ds(base, B_PER_W)], idx_all)

    rows = (rows0, rows1)
    sg = (sg0, sg1)
    sw = (sw0, sw1)

    def start_gather(j):
        return pltpu.async_copy(
            w_hbm.at[idx_all.at[pl.ds(j * CHUNK, CHUNK)]], rows[j & 1],
            sg[j & 1])

    def start_write(j):
        return pltpu.async_copy(
            rows[j & 1], out_hbm.at[pl.ds(base + j * CHUNK, CHUNK)],
            sw[j & 1])

    gathers = [None] * N_CHUNKS
    writes = [None] * N_CHUNKS
    for j in range(N_CHUNKS):
        if j >= 2:
            writes[j - 2].wait()
        gathers[j] = start_gather(j)
        if j >= 1:
            gathers[j - 1].wait()
            writes[j - 1] = start_write(j - 1)
    gathers[N_CHUNKS - 1].wait()
    writes[N_CHUNKS - 1] = start_write(N_CHUNKS - 1)
    writes[N_CHUNKS - 2].wait()
    writes[N_CHUNKS - 1].wait()


@jax.jit
def _emb_call(x_flat, weight):
    fn = functools.partial(
        pl.kernel,
        mesh=plsc.VectorSubcoreMesh(core_axis_name="c", subcore_axis_name="s"),
        out_type=jax.ShapeDtypeStruct((B_TOTAL, D), jnp.float32),
        scratch_types=[
            pltpu.VMEM((B_PER_W,), jnp.int32),
            pltpu.VMEM((CHUNK, D), jnp.float32),
            pltpu.VMEM((CHUNK, D), jnp.float32),
            pltpu.SemaphoreType.DMA,
            pltpu.SemaphoreType.DMA,
            pltpu.SemaphoreType.DMA,
            pltpu.SemaphoreType.DMA,
        ],
        compiler_params=pltpu.CompilerParams(use_tc_tiling_on_sc=False),
    )(_emb_body)
    return fn(x_flat, weight)


def kernel(x, weight):
    x_flat = x.reshape(-1).astype(jnp.int32)
    out = _emb_call(x_flat, weight)
    return out.reshape(x.shape + (weight.shape[1],))


# speedup vs baseline: 1.5173x; 1.0114x over previous
import functools

import jax
import jax.numpy as jnp
from jax import lax
from jax.experimental import pallas as pl
from jax.experimental.pallas import tpu as pltpu
from jax.experimental.pallas import tpu_sc as plsc

D = 32
B_TOTAL = 4096 * 200
NC = 2
NS = 16
NW = NC * NS
B_PER_W = B_TOTAL // NW
CHUNK = 1280
N_CHUNKS = B_PER_W // CHUNK


def _emb_body(x_hbm, w_hbm, out_hbm, idx_all, rows0, rows1,
              sg0, sg1, sw0, sw1):
    wid = lax.axis_index("s") * NC + lax.axis_index("c")
    base = wid * B_PER_W
    pltpu.sync_copy(x_hbm.at[pl.ds(base, B_PER_W)], idx_all)

    rows = (rows0, rows1)
    sg = (sg0, sg1)
    sw = (sw0, sw1)

    def start_gather(j):
        return pltpu.async_copy(
            w_hbm.at[idx_all.at[pl.ds(j * CHUNK, CHUNK)]], rows[j & 1],
            sg[j & 1])

    def start_write(j):
        return pltpu.async_copy(
            rows[j & 1], out_hbm.at[pl.ds(base + j * CHUNK, CHUNK)],
            sw[j & 1])

    gathers = [None] * N_CHUNKS
    writes = [None] * N_CHUNKS
    for j in range(N_CHUNKS):
        if j >= 2:
            writes[j - 2].wait()
        gathers[j] = start_gather(j)
        if j >= 1:
            gathers[j - 1].wait()
            writes[j - 1] = start_write(j - 1)
    gathers[N_CHUNKS - 1].wait()
    writes[N_CHUNKS - 1] = start_write(N_CHUNKS - 1)
    writes[N_CHUNKS - 2].wait()
    writes[N_CHUNKS - 1].wait()


@jax.jit
def _emb_call(x_flat, weight):
    fn = functools.partial(
        pl.kernel,
        mesh=plsc.VectorSubcoreMesh(core_axis_name="c", subcore_axis_name="s"),
        out_type=jax.ShapeDtypeStruct((B_TOTAL, D), jnp.float32),
        scratch_types=[
            pltpu.VMEM((B_PER_W,), jnp.int32),
            pltpu.VMEM((CHUNK, D), jnp.float32),
            pltpu.VMEM((CHUNK, D), jnp.float32),
            pltpu.SemaphoreType.DMA,
            pltpu.SemaphoreType.DMA,
            pltpu.SemaphoreType.DMA,
            pltpu.SemaphoreType.DMA,
        ],
        compiler_params=pltpu.CompilerParams(use_tc_tiling_on_sc=False),
    )(_emb_body)
    return fn(x_flat, weight)


def kernel(x, weight):
    x_flat = x.reshape(-1).astype(jnp.int32) * 4
    wp = jnp.pad(weight, ((0, 0), (0, 96))).reshape(4 * weight.shape[0], 32)
    out = _emb_call(x_flat, wp)
    return out.reshape(x.shape + (weight.shape[1],))


# speedup vs baseline: 1.6176x; 1.0661x over previous
import functools

import jax
import jax.numpy as jnp
from jax import lax
from jax.experimental import pallas as pl
from jax.experimental.pallas import tpu as pltpu
from jax.experimental.pallas import tpu_sc as plsc

D = 32
V = 1000000
B_TOTAL = 4096 * 200
NC = 2
NS = 16
NW = NC * NS

B_PER_W = B_TOTAL // NW
CHUNK = 1280
N_CHUNKS = B_PER_W // CHUNK

FQ = 80
F_PER_W = (B_TOTAL // 4) // NW
F_CHUNKS = F_PER_W // FQ


def _emb_body(x_hbm, w_hbm, out_hbm, idx_all, rows0, rows1,
              sg0, sg1, sw0, sw1):
    wid = lax.axis_index("s") * NC + lax.axis_index("c")
    base = wid * B_PER_W
    pltpu.sync_copy(x_hbm.at[pl.ds(base, B_PER_W)], idx_all)

    rows = (rows0, rows1)
    sg = (sg0, sg1)
    sw = (sw0, sw1)

    def start_gather(j):
        return pltpu.async_copy(
            w_hbm.at[idx_all.at[pl.ds(j * CHUNK, CHUNK)]], rows[j & 1],
            sg[j & 1])

    def start_write(j):
        return pltpu.async_copy(
            rows[j & 1], out_hbm.at[pl.ds(base + j * CHUNK, CHUNK)],
            sw[j & 1])

    gathers = [None] * N_CHUNKS
    writes = [None] * N_CHUNKS
    for j in range(N_CHUNKS):
        if j >= 2:
            writes[j - 2].wait()
        gathers[j] = start_gather(j)
        if j >= 1:
            gathers[j - 1].wait()
            writes[j - 1] = start_write(j - 1)
    gathers[N_CHUNKS - 1].wait()
    writes[N_CHUNKS - 1] = start_write(N_CHUNKS - 1)
    writes[N_CHUNKS - 2].wait()
    writes[N_CHUNKS - 1].wait()


@jax.jit
def _emb_call(x_flat, weight):
    fn = functools.partial(
        pl.kernel,
        mesh=plsc.VectorSubcoreMesh(core_axis_name="c", subcore_axis_name="s"),
        out_type=jax.ShapeDtypeStruct((B_TOTAL, D), jnp.float32),
        scratch_types=[
            pltpu.VMEM((B_PER_W,), jnp.int32),
            pltpu.VMEM((CHUNK, D), jnp.float32),
            pltpu.VMEM((CHUNK, D), jnp.float32),
            pltpu.SemaphoreType.DMA,
            pltpu.SemaphoreType.DMA,
            pltpu.SemaphoreType.DMA,
            pltpu.SemaphoreType.DMA,
        ],
        compiler_params=pltpu.CompilerParams(use_tc_tiling_on_sc=False),
    )(_emb_body)
    return fn(x_flat, weight)


def _fmt_body(l2_hbm, ow_hbm, a0, a1, b0, b1, sr0, sr1, sw0, sw1):
    wid = lax.axis_index("s") * NC + lax.axis_index("c")
    base2 = wid * F_PER_W

    bufa = (a0, a1)
    bufb = (b0, b1)
    srs = (sr0, sr1)
    sws = (sw0, sw1)

    def vec2(i, a, b):
        del i

        def body(k, carry):
            q0 = k * 8
            for r in range(8):
                q = q0 + r
                for c0 in range(0, 128, 16):
                    b[4 * q + c0 // 32, pl.ds(c0 % 32, 16)] = (
                        a[q, pl.ds(c0, 16)])
            return carry

        lax.fori_loop(0, FQ // 8, body, 0)

    def rd_t(i, bi):
        return pltpu.make_async_copy(
            l2_hbm.at[pl.ds(base2 + i * FQ, FQ), :], bufa[bi], srs[bi])

    def wr_t(i, bi):
        return pltpu.make_async_copy(
            bufb[bi],
            ow_hbm.at[pl.ds(4 * (base2 + i * FQ), 4 * FQ), :], sws[bi])

    rd_t(0, 0).start()

    def outer(k, carry):
        i0 = 2 * k
        i1 = i0 + 1
        rd_t(i1, 1).start()

        @pl.when(k > 0)
        def _():
            wr_t(i0 - 2, 0).wait()
        rd_t(i0, 0).wait()
        vec2(i0, bufa[0], bufb[0])
        wr_t(i0, 0).start()

        @pl.when(k < F_CHUNKS // 2 - 1)
        def _():
            rd_t(i0 + 2, 0).start()

        @pl.when(k > 0)
        def _():
            wr_t(i1 - 2, 1).wait()
        rd_t(i1, 1).wait()
        vec2(i1, bufa[1], bufb[1])
        wr_t(i1, 1).start()
        return carry

    lax.fori_loop(0, F_CHUNKS // 2, outer, 0)
    wr_t(F_CHUNKS - 2, 0).wait()
    wr_t(F_CHUNKS - 1, 1).wait()


@jax.jit
def _fmt_call(l2):
    fn = functools.partial(
        pl.kernel,
        mesh=plsc.VectorSubcoreMesh(core_axis_name="c", subcore_axis_name="s"),
        out_type=jax.ShapeDtypeStruct((B_TOTAL, D), jnp.float32),
        scratch_types=[
            pltpu.VMEM((FQ, 128), jnp.float32),
            pltpu.VMEM((FQ, 128), jnp.float32),
            pltpu.VMEM((4 * FQ, D), jnp.float32),
            pltpu.VMEM((4 * FQ, D), jnp.float32),
            pltpu.SemaphoreType.DMA,
            pltpu.SemaphoreType.DMA,
            pltpu.SemaphoreType.DMA,
            pltpu.SemaphoreType.DMA,
        ],
        compiler_params=pltpu.CompilerParams(use_tc_tiling_on_sc=True),
    )(_fmt_body)
    return fn(l2)


def kernel(x, weight):
    x_flat = x.reshape(-1).astype(jnp.int32) * 4
    wp = jnp.pad(weight, ((0, 0), (0, 96))).reshape(4 * V, D)
    lin = _emb_call(x_flat, wp)
    ow = _fmt_call(lin.reshape(B_TOTAL // 4, 128))
    return ow.reshape(x.shape + (weight.shape[1],))
